# baseline (speedup 1.0000x reference)
import functools

import jax
import jax.numpy as jnp
from jax import lax
from jax.experimental import pallas as pl
from jax.experimental.pallas import tpu as pltpu
from jax.experimental.pallas import tpu_sc as plsc

_N_NODES = 100000
_N_CLASSES = 16
_N_EDGES = 6400000

_NC = 2
_NS = 16
_NW = _NC * _NS
_CHUNK = 5120
_N_CHUNKS = _N_EDGES // _CHUNK
_NBUF = 3
_KMAX = -(-_N_CHUNKS // _NW)
_KMAX += (-_KMAX) % _NBUF
_UNROLL = 16
_TABLE_WORDS = _N_NODES // 4


def _argmax_body(xt_ref, out_ref):
    x = xt_ref[...]
    m = jnp.max(x, axis=0)
    idx = lax.broadcasted_iota(jnp.int32, x.shape, 0)
    cand = jnp.where(x == m[None, :], idx, _N_CLASSES)
    out_ref[...] = jnp.min(cand, axis=0).astype(jnp.int8)


def _compute_classes_packed(kg_onehot):
    xt = kg_onehot.T
    cls8 = pl.pallas_call(
        _argmax_body,
        out_shape=jax.ShapeDtypeStruct((_N_NODES,), jnp.int8),
    )(xt)
    return lax.bitcast_convert_type(
        cls8.reshape(_TABLE_WORDS, 4), jnp.int32)


def _sc_edge_body(cls_hbm, edge_hbm, bias_hbm, out_hbm,
                  table_v, ebuf0_v, ebuf1_v, ebuf2_v,
                  out0_v, out1_v, out2_v, bias_v,
                  in_sem0, in_sem1, in_sem2, out_sem0, out_sem1, out_sem2):
    c = lax.axis_index("c")
    s = lax.axis_index("s")
    wid = s * _NC + c
    nk = (_N_CHUNKS - wid + _NW - 1) // _NW
    ebuf = (ebuf0_v, ebuf1_v, ebuf2_v)
    outb = (out0_v, out1_v, out2_v)
    in_sems = (in_sem0, in_sem1, in_sem2)
    out_sems = (out_sem0, out_sem1, out_sem2)

    def in_copy(k, b):
        off = (wid + k * _NW) * _CHUNK
        return pltpu.make_async_copy(
            edge_hbm.at[:, pl.ds(off, _CHUNK)], ebuf[b], in_sems[b])

    def out_copy(k, b):
        off = (wid + k * _NW) * _CHUNK
        return pltpu.make_async_copy(
            outb[b], out_hbm.at[pl.ds(off, _CHUNK)], out_sems[b])

    @pl.when(0 < nk)
    def _():
        in_copy(0, 0).start()

    @pl.when(1 < nk)
    def _():
        in_copy(1, 1).start()

    pltpu.sync_copy(cls_hbm, table_v)
    pltpu.sync_copy(bias_hbm, bias_v)
    bias = bias_v[...]
    zero = jnp.zeros((16,), jnp.float32)
    lanes = lax.broadcasted_iota(jnp.int32, (16,), 0)
    row0 = jnp.zeros((16,), jnp.int32)
    row1 = jnp.ones((16,), jnp.int32)

    def compute(b):
        eb = ebuf[b]
        ob = outb[b]

        @plsc.parallel_loop(0, _CHUNK, step=16, unroll=_UNROLL)
        def inner(ii):
            col = lanes + ii
            sidx = plsc.load_gather(eb, [row0, col])
            didx = plsc.load_gather(eb, [row1, col])
            ws = plsc.load_gather(table_v, [sidx >> 2])
            wd = plsc.load_gather(table_v, [didx >> 2])
            cs = (ws >> ((sidx & 3) << 3)) & 255
            cd = (wd >> ((didx & 3) << 3)) & 255
            ob[pl.ds(ii, 16)] = jnp.where(cs == cd, bias, zero)

    @pl.loop(0, _KMAX, step=_NBUF)
    def ring(kb):
        for b in range(_NBUF):
            k = kb + b

            @pl.when(k + 2 < nk)
            def _():
                in_copy(k + 2, (b + 2) % _NBUF).start()

            @pl.when(k < nk)
            def _():
                in_copy(k, b).wait()

            @pl.when(jnp.logical_and(k >= _NBUF, k - _NBUF < nk))
            def _():
                out_copy(k - _NBUF, b).wait()

            @pl.when(k < nk)
            def _():
                compute(b)
                out_copy(k, b).start()

    for b in range(_NBUF):
        k = _KMAX - _NBUF + b

        @pl.when(k < nk)
        def _():
            out_copy(k, b).wait()


def _sc_edge_kernel(cls_packed, edge_index, bias_vec):
    mesh = plsc.VectorSubcoreMesh(core_axis_name="c", subcore_axis_name="s")
    f = functools.partial(
        pl.kernel,
        mesh=mesh,
        compiler_params=pltpu.CompilerParams(needs_layout_passes=False),
        out_type=jax.ShapeDtypeStruct((_N_EDGES,), jnp.float32),
        scratch_types=[
            pltpu.VMEM((_TABLE_WORDS,), jnp.int32),
            pltpu.VMEM((2, _CHUNK), jnp.int32),
            pltpu.VMEM((2, _CHUNK), jnp.int32),
            pltpu.VMEM((2, _CHUNK), jnp.int32),
            pltpu.VMEM((_CHUNK,), jnp.float32),
            pltpu.VMEM((_CHUNK,), jnp.float32),
            pltpu.VMEM((_CHUNK,), jnp.float32),
            pltpu.VMEM((16,), jnp.float32),
            pltpu.SemaphoreType.DMA,
            pltpu.SemaphoreType.DMA,
            pltpu.SemaphoreType.DMA,
            pltpu.SemaphoreType.DMA,
            pltpu.SemaphoreType.DMA,
            pltpu.SemaphoreType.DMA,
        ],
    )(_sc_edge_body)
    return f(cls_packed, edge_index, bias_vec)


def kernel(kg_onehot, edge_index, same_class_bias):
    cls_packed = _compute_classes_packed(kg_onehot)
    bias_vec = jnp.full((16,), same_class_bias, jnp.float32)
    return _sc_edge_kernel(cls_packed, edge_index, bias_vec)


# speedup vs baseline: 1.4112x; 1.4112x over previous
import functools

import jax
import jax.numpy as jnp
from jax import lax
from jax.experimental import pallas as pl
from jax.experimental.pallas import tpu as pltpu
from jax.experimental.pallas import tpu_sc as plsc

_N_NODES = 100000
_N_CLASSES = 16
_N_EDGES = 6400000

_NC = 2
_NS = 16
_NW = _NC * _NS
_CHUNK = 2560
_N_CHUNKS = _N_EDGES // _CHUNK
_NBUF = 3
_KMAX = -(-_N_CHUNKS // _NW)
_KMAX += (-_KMAX) % _NBUF
_UNROLL = 16
_TABLE_WORDS = _N_NODES


def _argmax_body(xt_ref, out_ref):
    x = xt_ref[...]
    m = jnp.max(x, axis=0)
    idx = lax.broadcasted_iota(jnp.int32, x.shape, 0)
    cand = jnp.where(x == m[None, :], idx, _N_CLASSES)
    out_ref[...] = jnp.min(cand, axis=0)


def _compute_classes_packed(kg_onehot):
    xt = kg_onehot.T
    return pl.pallas_call(
        _argmax_body,
        out_shape=jax.ShapeDtypeStruct((_N_NODES,), jnp.int32),
    )(xt)


def _sc_edge_body(cls_hbm, edge_hbm, bias_hbm, out_hbm,
                  table_v, ebuf0_v, ebuf1_v, ebuf2_v,
                  out0_v, out1_v, out2_v, bias_v,
                  in_sem0, in_sem1, in_sem2, out_sem0, out_sem1, out_sem2):
    c = lax.axis_index("c")
    s = lax.axis_index("s")
    wid = s * _NC + c
    nk = (_N_CHUNKS - wid + _NW - 1) // _NW
    ebuf = (ebuf0_v, ebuf1_v, ebuf2_v)
    outb = (out0_v, out1_v, out2_v)
    in_sems = (in_sem0, in_sem1, in_sem2)
    out_sems = (out_sem0, out_sem1, out_sem2)

    def in_copy(k, b):
        off = (wid + k * _NW) * _CHUNK
        return pltpu.make_async_copy(
            edge_hbm.at[:, pl.ds(off, _CHUNK)], ebuf[b], in_sems[b])

    def out_copy(k, b):
        off = (wid + k * _NW) * _CHUNK
        return pltpu.make_async_copy(
            outb[b], out_hbm.at[pl.ds(off, _CHUNK)], out_sems[b])

    @pl.when(0 < nk)
    def _():
        in_copy(0, 0).start()

    @pl.when(1 < nk)
    def _():
        in_copy(1, 1).start()

    pltpu.sync_copy(cls_hbm, table_v)
    pltpu.sync_copy(bias_hbm, bias_v)
    bias = bias_v[...]
    zero = jnp.zeros((16,), jnp.float32)
    lanes = lax.broadcasted_iota(jnp.int32, (16,), 0)
    row0 = jnp.zeros((16,), jnp.int32)
    row1 = jnp.ones((16,), jnp.int32)

    def compute(b):
        eb = ebuf[b]
        ob = outb[b]

        @plsc.parallel_loop(0, _CHUNK, step=16, unroll=_UNROLL)
        def inner(ii):
            col = lanes + ii
            sidx = plsc.load_gather(eb, [row0, col])
            didx = plsc.load_gather(eb, [row1, col])
            cs = plsc.load_gather(table_v, [sidx])
            cd = plsc.load_gather(table_v, [didx])
            ob[pl.ds(ii, 16)] = jnp.where(cs == cd, bias, zero)

    @pl.loop(0, _KMAX, step=_NBUF)
    def ring(kb):
        for b in range(_NBUF):
            k = kb + b

            @pl.when(k + 2 < nk)
            def _():
                in_copy(k + 2, (b + 2) % _NBUF).start()

            @pl.when(k < nk)
            def _():
                in_copy(k, b).wait()

            @pl.when(jnp.logical_and(k >= _NBUF, k - _NBUF < nk))
            def _():
                out_copy(k - _NBUF, b).wait()

            @pl.when(k < nk)
            def _():
                compute(b)
                out_copy(k, b).start()

    for b in range(_NBUF):
        k = _KMAX - _NBUF + b

        @pl.when(k < nk)
        def _():
            out_copy(k, b).wait()


def _sc_edge_kernel(cls_packed, edge_index, bias_vec):
    mesh = plsc.VectorSubcoreMesh(core_axis_name="c", subcore_axis_name="s")
    f = functools.partial(
        pl.kernel,
        mesh=mesh,
        compiler_params=pltpu.CompilerParams(needs_layout_passes=False),
        out_type=jax.ShapeDtypeStruct((_N_EDGES,), jnp.float32),
        scratch_types=[
            pltpu.VMEM((_TABLE_WORDS,), jnp.int32),
            pltpu.VMEM((2, _CHUNK), jnp.int32),
            pltpu.VMEM((2, _CHUNK), jnp.int32),
            pltpu.VMEM((2, _CHUNK), jnp.int32),
            pltpu.VMEM((_CHUNK,), jnp.float32),
            pltpu.VMEM((_CHUNK,), jnp.float32),
            pltpu.VMEM((_CHUNK,), jnp.float32),
            pltpu.VMEM((16,), jnp.float32),
            pltpu.SemaphoreType.DMA,
            pltpu.SemaphoreType.DMA,
            pltpu.SemaphoreType.DMA,
            pltpu.SemaphoreType.DMA,
            pltpu.SemaphoreType.DMA,
            pltpu.SemaphoreType.DMA,
        ],
    )(_sc_edge_body)
    return f(cls_packed, edge_index, bias_vec)


def kernel(kg_onehot, edge_index, same_class_bias):
    cls_packed = _compute_classes_packed(kg_onehot)
    bias_vec = jnp.full((16,), same_class_bias, jnp.float32)
    return _sc_edge_kernel(cls_packed, edge_index, bias_vec)
